# baseline (speedup 1.0000x reference)
import math

import jax
import jax.numpy as jnp
from jax import lax
from jax.experimental import pallas as pl

_P = 8
_PP = _P * _P * _P
_INV_SQRT_2PI_CUBED = 1.0 / (2.0 * math.pi) ** 1.5


def _raster_body(sig_ref, tail_ref, time_ref, charge_ref, sp_ref, ns_ref,
                 out_ref, off_ref):
    f32 = jnp.float32
    sig0 = sig_ref[:, 0:1]
    sig1 = sig_ref[:, 1:2]
    sig2 = sig_ref[:, 2:3]
    c0 = tail_ref[:, 1:2]
    c1 = tail_ref[:, 2:3]
    c2 = time_ref[:, 0:1]
    s0 = sp_ref[0:1, 0:1]
    s1 = sp_ref[0:1, 1:2]
    s2 = sp_ref[0:1, 2:3]
    ns = ns_ref[0:1, 0:1]

    lo0 = jnp.floor((c0 - ns * sig0) / s0)
    lo1 = jnp.floor((c1 - ns * sig1) / s1)
    lo2 = jnp.floor((c2 - ns * sig2) / s2)
    off_ref[:, :] = jnp.concatenate(
        [lo0.astype(jnp.int32), lo1.astype(jnp.int32), lo2.astype(jnp.int32)],
        axis=1)

    inv0 = 1.0 / sig0
    inv1 = 1.0 / sig1
    inv2 = 1.0 / sig2
    m0 = s0 * inv0
    m1 = s1 * inv1
    m2 = s2 * inv2
    b0 = ((lo0 + 0.5) * s0 - c0) * inv0
    b1 = ((lo1 + 0.5) * s1 - c1) * inv1
    b2 = ((lo2 + 0.5) * s2 - c2) * inv2
    amp = (charge_ref[:, 0:1] * (s0 * s1 * s2) * _INV_SQRT_2PI_CUBED
           * (inv0 * inv1 * inv2))

    blk = out_ref.shape[0]
    q = lax.broadcasted_iota(jnp.int32, (blk, 128), 1)
    jf = ((q >> 3) & 7).astype(f32)
    kf = (q & 7).astype(f32)
    z1 = b1 + m1 * jf
    z2 = b2 + m2 * kf
    e12 = amp * jnp.exp(-0.5 * (z1 * z1 + z2 * z2))
    hi = q >= 64
    for t in range(4):
        z0a = b0 + m0 * f32(2 * t)
        z0b = b0 + m0 * f32(2 * t + 1)
        e0a = jnp.exp(-0.5 * z0a * z0a)
        e0b = jnp.exp(-0.5 * z0b * z0b)
        g0 = jnp.where(hi, e0b, e0a)
        out_ref[:, t * 128:(t + 1) * 128] = g0 * e12


def _pick_block(n):
    for b in (800, 1000, 2000, 400, 320, 200, 160, 128, 80, 64, 40, 32, 16, 8):
        if n % b == 0 and b % 8 == 0:
            return b
    return None


def _raster_call(sigma, time, charge, tail, grid_spacing, nsigma, block):
    n = sigma.shape[0]
    grid = n // block
    rasters, offsets = pl.pallas_call(
        _raster_body,
        grid=(grid,),
        in_specs=[
            pl.BlockSpec((block, 3), lambda i: (i, 0)),
            pl.BlockSpec((block, 3), lambda i: (i, 0)),
            pl.BlockSpec((block, 1), lambda i: (i, 0)),
            pl.BlockSpec((block, 1), lambda i: (i, 0)),
            pl.BlockSpec((1, 3), lambda i: (0, 0)),
            pl.BlockSpec((1, 1), lambda i: (0, 0)),
        ],
        out_specs=[
            pl.BlockSpec((block, _PP), lambda i: (i, 0)),
            pl.BlockSpec((block, 3), lambda i: (i, 0)),
        ],
        out_shape=[
            jax.ShapeDtypeStruct((n, _PP), jnp.float32),
            jax.ShapeDtypeStruct((n, 3), jnp.int32),
        ],
    )(sigma, tail, time.reshape(n, 1), charge.reshape(n, 1),
      grid_spacing.reshape(1, 3), jnp.reshape(nsigma, (1, 1)))
    return rasters, offsets


def kernel(sigma, time, charge, tail, grid_spacing, nsigma):
    n = sigma.shape[0]
    block = _pick_block(n)
    if block is None:
        block = 8
        npad = -(-n // block) * block
        pad = npad - n
        rasters, offsets = _raster_call(
            jnp.pad(sigma, ((0, pad), (0, 0)), constant_values=1.0),
            jnp.pad(time, (0, pad)), jnp.pad(charge, (0, pad)),
            jnp.pad(tail, ((0, pad), (0, 0))), grid_spacing, nsigma, block)
        rasters = rasters[:n]
        offsets = offsets[:n]
    else:
        rasters, offsets = _raster_call(sigma, time, charge, tail,
                                        grid_spacing, nsigma, block)
    return rasters.reshape(n, _P, _P, _P), offsets


# speedup vs baseline: 2.3030x; 2.3030x over previous
import math

import jax
import jax.numpy as jnp
from jax import lax
from jax.experimental import pallas as pl

_P = 8
_PP = _P * _P * _P
_INV_SQRT_2PI_CUBED = 1.0 / (2.0 * math.pi) ** 1.5


def _raster_body(par_ref, sp_ref, ns_ref, out_ref, off_ref):
    f32 = jnp.float32
    sig0 = par_ref[0, 0:1, :]
    sig1 = par_ref[0, 1:2, :]
    sig2 = par_ref[0, 2:3, :]
    c0 = par_ref[0, 3:4, :]
    c1 = par_ref[0, 4:5, :]
    c2 = par_ref[0, 5:6, :]
    chg = par_ref[0, 6:7, :]
    s0 = sp_ref[0:1, 0:1]
    s1 = sp_ref[0:1, 1:2]
    s2 = sp_ref[0:1, 2:3]
    ns = ns_ref[0:1, 0:1]

    lo0 = jnp.floor((c0 - ns * sig0) / s0)
    lo1 = jnp.floor((c1 - ns * sig1) / s1)
    lo2 = jnp.floor((c2 - ns * sig2) / s2)
    off_ref[0, 0:1, :] = lo0.astype(jnp.int32)
    off_ref[0, 1:2, :] = lo1.astype(jnp.int32)
    off_ref[0, 2:3, :] = lo2.astype(jnp.int32)

    inv0 = 1.0 / sig0
    inv1 = 1.0 / sig1
    inv2 = 1.0 / sig2
    m0 = s0 * inv0
    m1 = s1 * inv1
    m2 = s2 * inv2
    b0 = ((lo0 + 0.5) * s0 - c0) * inv0
    b1 = ((lo1 + 0.5) * s1 - c1) * inv1
    b2 = ((lo2 + 0.5) * s2 - c2) * inv2
    amp = chg * (s0 * s1 * s2) * _INV_SQRT_2PI_CUBED * (inv0 * inv1 * inv2)

    blk = par_ref.shape[2]
    ii = lax.broadcasted_iota(jnp.int32, (_P, blk), 0).astype(f32)
    z0 = b0 + m0 * ii
    e0 = amp * jnp.exp(-0.5 * (z0 * z0))
    rr = lax.broadcasted_iota(jnp.int32, (_P * _P, blk), 0)
    jf = (rr >> 3).astype(f32)
    kf = (rr & 7).astype(f32)
    z1 = b1 + m1 * jf
    z2 = b2 + m2 * kf
    e12 = jnp.exp(-0.5 * (z1 * z1 + z2 * z2))

    pp = lax.broadcasted_iota(jnp.int32, (_P, _PP), 1)
    s0m = (pp >> 6 == lax.broadcasted_iota(jnp.int32, (_P, _PP), 0)).astype(f32)
    pp2 = lax.broadcasted_iota(jnp.int32, (_P * _P, _PP), 1)
    s12m = ((pp2 & 63) == lax.broadcasted_iota(jnp.int32, (_P * _P, _PP), 0)
            ).astype(f32)

    dn = (((0,), (0,)), ((), ()))
    g0 = lax.dot_general(e0, s0m, dn, preferred_element_type=f32)
    g12 = lax.dot_general(e12, s12m, dn, preferred_element_type=f32)
    out_ref[:, :] = g0 * g12


def _pick_block(n):
    for b in (800, 1000, 2000, 400, 320, 200, 160, 128, 80, 64, 40, 32, 16, 8):
        if n % b == 0 and b % 8 == 0:
            return b
    return None


def _raster_call(params, grid_spacing, nsigma, block):
    n = params.shape[1]
    grid = n // block
    params3 = params.reshape(8, grid, block).transpose(1, 0, 2)
    rasters, offsets = pl.pallas_call(
        _raster_body,
        grid=(grid,),
        in_specs=[
            pl.BlockSpec((1, 8, block), lambda i: (i, 0, 0)),
            pl.BlockSpec((1, 3), lambda i: (0, 0)),
            pl.BlockSpec((1, 1), lambda i: (0, 0)),
        ],
        out_specs=[
            pl.BlockSpec((block, _PP), lambda i: (i, 0)),
            pl.BlockSpec((1, 3, block), lambda i: (i, 0, 0)),
        ],
        out_shape=[
            jax.ShapeDtypeStruct((n, _PP), jnp.float32),
            jax.ShapeDtypeStruct((grid, 3, block), jnp.int32),
        ],
    )(params3, grid_spacing.reshape(1, 3), jnp.reshape(nsigma, (1, 1)))
    offsets = offsets.transpose(0, 2, 1).reshape(n, 3)
    return rasters, offsets


def kernel(sigma, time, charge, tail, grid_spacing, nsigma):
    n = sigma.shape[0]
    params = jnp.stack(
        [sigma[:, 0], sigma[:, 1], sigma[:, 2],
         tail[:, 1], tail[:, 2], time, charge,
         jnp.zeros_like(time)], axis=0)
    block = _pick_block(n)
    if block is None:
        block = 8
        npad = -(-n // block) * block
        params = jnp.pad(params, ((0, 0), (0, npad - n)), constant_values=1.0)
    rasters, offsets = _raster_call(params, grid_spacing, nsigma, block)
    rasters = rasters[:n] if rasters.shape[0] != n else rasters
    offsets = offsets[:n] if offsets.shape[0] != n else offsets
    return rasters.reshape(n, _P, _P, _P), offsets


# speedup vs baseline: 7.6419x; 3.3183x over previous
import math

import jax
import jax.numpy as jnp
from jax import lax
from jax.experimental import pallas as pl
from jax.experimental.pallas import tpu as pltpu

_P = 8
_INV_SQRT_2PI_CUBED = 1.0 / (2.0 * math.pi) ** 1.5


def _raster_body(par_ref, sp_ref, ns_ref, out_ref, off_ref,
                 e0_ref, e1_ref, e2_ref):
    f32 = jnp.float32
    i = pl.program_id(0)
    j = pl.program_id(1)

    @pl.when((i == 0) & (j == 0))
    def _build_tables():
        sig0 = par_ref[0:1, :]
        sig1 = par_ref[1:2, :]
        sig2 = par_ref[2:3, :]
        c0 = par_ref[3:4, :]
        c1 = par_ref[4:5, :]
        c2 = par_ref[5:6, :]
        chg = par_ref[6:7, :]
        s0 = sp_ref[0:1, 0:1]
        s1 = sp_ref[0:1, 1:2]
        s2 = sp_ref[0:1, 2:3]
        ns = ns_ref[0:1, 0:1]

        lo0 = jnp.floor((c0 - ns * sig0) / s0)
        lo1 = jnp.floor((c1 - ns * sig1) / s1)
        lo2 = jnp.floor((c2 - ns * sig2) / s2)
        off_ref[0:1, :] = lo0.astype(jnp.int32)
        off_ref[1:2, :] = lo1.astype(jnp.int32)
        off_ref[2:3, :] = lo2.astype(jnp.int32)

        inv0 = 1.0 / sig0
        inv1 = 1.0 / sig1
        inv2 = 1.0 / sig2
        m0 = s0 * inv0
        m1 = s1 * inv1
        m2 = s2 * inv2
        b0 = ((lo0 + 0.5) * s0 - c0) * inv0
        b1 = ((lo1 + 0.5) * s1 - c1) * inv1
        b2 = ((lo2 + 0.5) * s2 - c2) * inv2
        amp = (chg * (s0 * s1 * s2) * _INV_SQRT_2PI_CUBED
               * (inv0 * inv1 * inv2))

        n = par_ref.shape[1]
        ii = lax.broadcasted_iota(jnp.int32, (_P, n), 0).astype(f32)
        z0 = b0 + m0 * ii
        z1 = b1 + m1 * ii
        z2 = b2 + m2 * ii
        e0_ref[:, :] = amp * jnp.exp(-0.5 * (z0 * z0))
        e1_ref[:, :] = jnp.exp(-0.5 * (z1 * z1))
        e2_ref[:, :] = jnp.exp(-0.5 * (z2 * z2))

    g01 = e0_ref[pl.ds(i, 1), :] * e1_ref[pl.ds(j, 1), :]
    out_ref[0, 0, :, :] = e2_ref[:, :] * g01


def kernel(sigma, time, charge, tail, grid_spacing, nsigma):
    n = sigma.shape[0]
    params = jnp.stack(
        [sigma[:, 0], sigma[:, 1], sigma[:, 2],
         tail[:, 1], tail[:, 2], time, charge,
         jnp.zeros_like(time)], axis=0)
    rasters, offsets = pl.pallas_call(
        _raster_body,
        grid=(_P, _P),
        in_specs=[
            pl.BlockSpec((8, n), lambda i, j: (0, 0)),
            pl.BlockSpec((1, 3), lambda i, j: (0, 0)),
            pl.BlockSpec((1, 1), lambda i, j: (0, 0)),
        ],
        out_specs=[
            pl.BlockSpec((1, 1, _P, n), lambda i, j: (i, j, 0, 0)),
            pl.BlockSpec((3, n), lambda i, j: (0, 0)),
        ],
        out_shape=[
            jax.ShapeDtypeStruct((_P, _P, _P, n), jnp.float32),
            jax.ShapeDtypeStruct((3, n), jnp.int32),
        ],
        scratch_shapes=[
            pltpu.VMEM((_P, n), jnp.float32),
            pltpu.VMEM((_P, n), jnp.float32),
            pltpu.VMEM((_P, n), jnp.float32),
        ],
    )(params, grid_spacing.reshape(1, 3), jnp.reshape(nsigma, (1, 1)))
    return rasters.transpose(3, 0, 1, 2), offsets.T
